# Initial kernel scaffold; baseline (speedup 1.0000x reference)
import jax
import jax.numpy as jnp
from jax.experimental import pallas as pl


def kernel(x, edge_attr, cart_dist, gate_W1, gate_b1, gate_W2, gate_b2, aggr_W1, aggr_b1, aggr_W2, aggr_b2, bn_gamma, bn_beta, gn_alpha, gn_gamma, gn_beta, edge_index, batch_ids):
    raise NotImplementedError("write your pallas kernel here")


# speedup vs baseline: 3.2368x; 3.2368x over previous
import functools

import jax
import jax.numpy as jnp
from jax import lax
from jax.experimental import pallas as pl
from jax.experimental.pallas import tpu as pltpu
from jax.experimental.pallas import tpu_sc as plsc

RADIUS = 5.0
G = 64
EB = 2560
GCH = 400
SCH = 200


def _sc_gather(h, dst, src):
    N, D = h.shape
    E = dst.shape[0]
    mesh = plsc.VectorSubcoreMesh(core_axis_name="c", subcore_axis_name="s")
    NW = 32
    per_w = E // NW
    n_ch = per_w // GCH

    @functools.partial(
        pl.kernel,
        out_type=(jax.ShapeDtypeStruct((E, D), jnp.float32),
                  jax.ShapeDtypeStruct((E, D), jnp.float32)),
        mesh=mesh,
        scratch_types=[pltpu.VMEM((GCH,), jnp.int32),
                       pltpu.VMEM((GCH, D), jnp.float32),
                       pltpu.SemaphoreType.DMA],
    )
    def gk(h_hbm, dst_hbm, src_hbm, oi_hbm, oj_hbm, idx_v, rows_v, sem):
        wid = lax.axis_index("s") * 2 + lax.axis_index("c")
        for idx_hbm, out_hbm in ((dst_hbm, oi_hbm), (src_hbm, oj_hbm)):
            def body(j, _, idx_hbm=idx_hbm, out_hbm=out_hbm):
                base = wid * per_w + j * GCH
                pltpu.sync_copy(idx_hbm.at[pl.ds(base, GCH)], idx_v)
                pltpu.async_copy(h_hbm.at[idx_v], rows_v, sem).wait()
                pltpu.sync_copy(rows_v, out_hbm.at[pl.ds(base, GCH)])
                return 0
            lax.fori_loop(0, n_ch, body, 0)

    return gk(h, dst, src)


def _sc_scatter(sm, sig, dst, zeros_nd):
    E, D = sm.shape
    NP = zeros_nd.shape[0]
    mesh = plsc.VectorSubcoreMesh(core_axis_name="c", subcore_axis_name="s")
    per_t = E // 16
    n_ch = per_t // SCH
    rows_t = NP // 16

    @functools.partial(
        pl.kernel,
        out_type=(jax.ShapeDtypeStruct((NP, D), jnp.float32),
                  jax.ShapeDtypeStruct((NP, D), jnp.float32)),
        mesh=mesh,
        scratch_types=[pltpu.VMEM((SCH,), jnp.int32),
                       pltpu.VMEM((SCH, D), jnp.float32),
                       pltpu.VMEM_SHARED((NP, D), jnp.float32)],
    )
    def sk(sm_hbm, sig_hbm, dst_hbm, z_hbm, num_hbm, den_hbm,
           idx_v, vals_v, acc_sh):
        c = lax.axis_index("c")
        s = lax.axis_index("s")
        pltpu.sync_copy(z_hbm.at[pl.ds(s * rows_t, rows_t)],
                        acc_sh.at[pl.ds(s * rows_t, rows_t)])
        plsc.subcore_barrier()

        def body(j, _):
            base = s * per_t + j * SCH
            pltpu.sync_copy(dst_hbm.at[pl.ds(base, SCH)], idx_v)

            @pl.when(c == 0)
            def _():
                pltpu.sync_copy(sm_hbm.at[pl.ds(base, SCH)], vals_v)

            @pl.when(c == 1)
            def _():
                pltpu.sync_copy(sig_hbm.at[pl.ds(base, SCH)], vals_v)

            pltpu.sync_copy(vals_v, acc_sh.at[idx_v], add=True)
            return 0

        lax.fori_loop(0, n_ch, body, 0)
        plsc.subcore_barrier()

        @pl.when(c == 0)
        def _():
            pltpu.sync_copy(acc_sh.at[pl.ds(s * rows_t, rows_t)],
                            num_hbm.at[pl.ds(s * rows_t, rows_t)])

        @pl.when(c == 1)
        def _():
            pltpu.sync_copy(acc_sh.at[pl.ds(s * rows_t, rows_t)],
                            den_hbm.at[pl.ds(s * rows_t, rows_t)])

    return sk(sm, sig, dst, zeros_nd)


def _silu(v):
    return v / (1.0 + jnp.exp(-v))


def _tc_mlp(xi, xj, e, w1, b1, w2g, b2g, w2a, b2a):
    E, D = xi.shape
    n_blk = E // EB

    def body(xi_ref, xj_ref, e_ref, w1_ref, b1_ref, w2g_ref, b2g_ref,
             w2a_ref, b2a_ref, gate_ref, msg_ref, stats_ref, acc_ref):
        i = pl.program_id(0)

        @pl.when(i == 0)
        def _():
            acc_ref[...] = jnp.zeros_like(acc_ref)

        hid = (jnp.dot(xi_ref[...], w1_ref[0:D, :],
                       preferred_element_type=jnp.float32)
               + jnp.dot(xj_ref[...], w1_ref[D:2 * D, :],
                         preferred_element_type=jnp.float32)
               + jnp.dot(e_ref[...], w1_ref[2 * D:3 * D, :],
                         preferred_element_type=jnp.float32)
               + b1_ref[...])
        hid = _silu(hid)
        gate = jnp.dot(hid[:, :D], w2g_ref[...],
                       preferred_element_type=jnp.float32) + b2g_ref[...]
        msg = jnp.dot(hid[:, D:], w2a_ref[...],
                      preferred_element_type=jnp.float32) + b2a_ref[...]
        gate_ref[...] = gate
        msg_ref[...] = msg
        acc_ref[0:1, :] = acc_ref[0:1, :] + jnp.sum(gate, axis=0,
                                                    keepdims=True)
        acc_ref[1:2, :] = acc_ref[1:2, :] + jnp.sum(gate * gate, axis=0,
                                                    keepdims=True)
        stats_ref[...] = acc_ref[...]

    const = lambda i: (0, 0)
    return pl.pallas_call(
        body,
        grid=(n_blk,),
        in_specs=[
            pl.BlockSpec((EB, D), lambda i: (i, 0)),
            pl.BlockSpec((EB, D), lambda i: (i, 0)),
            pl.BlockSpec((EB, D), lambda i: (i, 0)),
            pl.BlockSpec((3 * D, 2 * D), const),
            pl.BlockSpec((1, 2 * D), const),
            pl.BlockSpec((D, D), const),
            pl.BlockSpec((1, D), const),
            pl.BlockSpec((D, D), const),
            pl.BlockSpec((1, D), const),
        ],
        out_specs=[
            pl.BlockSpec((EB, D), lambda i: (i, 0)),
            pl.BlockSpec((EB, D), lambda i: (i, 0)),
            pl.BlockSpec((8, D), const),
        ],
        out_shape=[
            jax.ShapeDtypeStruct((E, D), jnp.float32),
            jax.ShapeDtypeStruct((E, D), jnp.float32),
            jax.ShapeDtypeStruct((8, D), jnp.float32),
        ],
        scratch_shapes=[pltpu.VMEM((8, D), jnp.float32)],
    )(xi, xj, e, w1, b1, w2g, b2g, w2a, b2a)


def _tc_norm(gate_pre, msg, cart, stats, gamma, beta):
    E, D = gate_pre.shape
    n_blk = E // EB
    inv_e = 1.0 / E

    rows = EB // 128

    def body(g_ref, m_ref, c_ref, st_ref, ga_ref, be_ref,
             gn_ref, sm_ref, sg_ref):
        mu = st_ref[0:1, :] * inv_e
        var = st_ref[1:2, :] * inv_e - mu * mu
        rstd = lax.rsqrt(var + 1e-5)
        gate_n = (g_ref[...] - mu) * rstd * ga_ref[...] + be_ref[...]
        d = c_ref[0]
        env = 0.5 * (jnp.cos(jnp.pi * d / RADIUS) + 1.0)
        env = jnp.where(d < RADIUS, env, 0.0)
        ii = lax.broadcasted_iota(jnp.int32, (EB, rows), 0)
        rr = lax.broadcasted_iota(jnp.int32, (EB, rows), 1)
        sel = (ii // 128 == rr).astype(jnp.float32)
        x = jnp.dot(sel, env, preferred_element_type=jnp.float32)
        ic = lax.broadcasted_iota(jnp.int32, (EB, 128), 0) % 128
        cc = lax.broadcasted_iota(jnp.int32, (EB, 128), 1)
        env_col = jnp.sum(jnp.where(ic == cc, x, 0.0), axis=1,
                          keepdims=True)
        sig = env_col / (1.0 + jnp.exp(-gate_n))
        gn_ref[...] = gate_n
        sg_ref[...] = sig
        sm_ref[...] = sig * m_ref[...]

    const = lambda i: (0, 0)
    return pl.pallas_call(
        body,
        grid=(n_blk,),
        in_specs=[
            pl.BlockSpec((EB, D), lambda i: (i, 0)),
            pl.BlockSpec((EB, D), lambda i: (i, 0)),
            pl.BlockSpec((1, EB // 128, 128), lambda i: (i, 0, 0)),
            pl.BlockSpec((8, D), const),
            pl.BlockSpec((1, D), const),
            pl.BlockSpec((1, D), const),
        ],
        out_specs=[
            pl.BlockSpec((EB, D), lambda i: (i, 0)),
            pl.BlockSpec((EB, D), lambda i: (i, 0)),
            pl.BlockSpec((EB, D), lambda i: (i, 0)),
        ],
        out_shape=[
            jax.ShapeDtypeStruct((E, D), jnp.float32),
            jax.ShapeDtypeStruct((E, D), jnp.float32),
            jax.ShapeDtypeStruct((E, D), jnp.float32),
        ],
    )(gate_pre, msg, cart, stats, gamma, beta)


def _tc_node(num, den, h, bids_row, bids_col, alpha, gamma, beta):
    N, D = h.shape

    def body(num_ref, den_ref, h_ref, br_ref, bc_ref, al_ref, ga_ref, be_ref,
             out_ref):
        xnew = _silu(num_ref[...] / (den_ref[...] + 1e-6)) + h_ref[...]
        oh = (br_ref[...] == lax.broadcasted_iota(jnp.int32, (G, N), 0)
              ).astype(jnp.float32)
        oht = (bc_ref[...] == lax.broadcasted_iota(jnp.int32, (N, G), 1)
               ).astype(jnp.float32)
        cnt = jnp.maximum(jnp.sum(oh, axis=1, keepdims=True), 1.0)
        gmean = jnp.dot(oh, xnew, preferred_element_type=jnp.float32) / cnt
        xc = xnew - al_ref[...] * jnp.dot(
            oht, gmean, preferred_element_type=jnp.float32)
        gvar = jnp.dot(oh, xc * xc, preferred_element_type=jnp.float32) / cnt
        gv = jnp.dot(oht, gvar, preferred_element_type=jnp.float32)
        hn = xc * lax.rsqrt(gv + 1e-5) * ga_ref[...] + be_ref[...]
        out_ref[...] = h_ref[...] + hn

    const = lambda i: (0, 0)
    return pl.pallas_call(
        body,
        grid=(1,),
        in_specs=[
            pl.BlockSpec((N, D), const),
            pl.BlockSpec((N, D), const),
            pl.BlockSpec((N, D), const),
            pl.BlockSpec((1, N), const),
            pl.BlockSpec((N, 1), const),
            pl.BlockSpec((1, D), const),
            pl.BlockSpec((1, D), const),
            pl.BlockSpec((1, D), const),
        ],
        out_specs=pl.BlockSpec((N, D), const),
        out_shape=jax.ShapeDtypeStruct((N, D), jnp.float32),
    )(num, den, h, bids_row, bids_col, alpha, gamma, beta)


def kernel(x, edge_attr, cart_dist, gate_W1, gate_b1, gate_W2, gate_b2,
           aggr_W1, aggr_b1, aggr_W2, aggr_b2, bn_gamma, bn_beta,
           gn_alpha, gn_gamma, gn_beta, edge_index, batch_ids):
    N, D = x.shape
    E = edge_attr.shape[0]
    L = gate_W1.shape[0]
    src = edge_index[0]
    dst = edge_index[1]
    cart = cart_dist.reshape(E // EB, EB // 128, 128)
    bids_row = batch_ids.reshape(1, N)
    bids_col = batch_ids.reshape(N, 1)
    n_pad = ((N + 127) // 128) * 128
    zeros_nd = jnp.zeros((n_pad, D), jnp.float32)

    h = x
    e = edge_attr
    for l in range(L):
        w1 = jnp.concatenate([gate_W1[l], aggr_W1[l]], axis=1)
        b1 = jnp.concatenate([gate_b1[l], aggr_b1[l]])[None, :]
        xi, xj = _sc_gather(h, dst, src)
        gate_pre, msg, stats = _tc_mlp(
            xi, xj, e, w1, b1,
            gate_W2[l], gate_b2[l][None, :], aggr_W2[l], aggr_b2[l][None, :])
        gate_n, sm, sig = _tc_norm(
            gate_pre, msg, cart, stats,
            bn_gamma[l][None, :], bn_beta[l][None, :])
        num, den = _sc_scatter(sm, sig, dst, zeros_nd)
        h = _tc_node(num, den, h, bids_row, bids_col,
                     gn_alpha[l][None, :], gn_gamma[l][None, :],
                     gn_beta[l][None, :])
        e = gate_n
    return h


# speedup vs baseline: 3.5471x; 1.0959x over previous
import functools

import jax
import jax.numpy as jnp
from jax import lax
from jax.experimental import pallas as pl
from jax.experimental.pallas import tpu as pltpu
from jax.experimental.pallas import tpu_sc as plsc

RADIUS = 5.0
G = 64
EB = 2560
GCH = 80
SCH = 80


def _sc_gather(h, dst, src):
    NP, D = h.shape
    E = dst.shape[0]
    mesh = plsc.VectorSubcoreMesh(core_axis_name="c", subcore_axis_name="s")
    per_w = E // 32
    n_ch = per_w // GCH
    rows_t = NP // 16

    @functools.partial(
        pl.kernel,
        out_type=(jax.ShapeDtypeStruct((E, D), jnp.float32),
                  jax.ShapeDtypeStruct((E, D), jnp.float32)),
        mesh=mesh,
        scratch_types=[pltpu.VMEM((per_w,), jnp.int32),
                       pltpu.VMEM((per_w,), jnp.int32),
                       pltpu.VMEM((GCH, D), jnp.float32),
                       pltpu.VMEM((GCH, D), jnp.float32),
                       pltpu.VMEM_SHARED((NP, D), jnp.float32),
                       pltpu.SemaphoreType.DMA,
                       pltpu.SemaphoreType.DMA,
                       pltpu.SemaphoreType.DMA,
                       pltpu.SemaphoreType.DMA],
    )
    def gk(h_hbm, dst_hbm, src_hbm, oi_hbm, oj_hbm,
           idxd_v, idxs_v, buf0, buf1, h_sh, gsem0, gsem1, ssem0, ssem1):
        c = lax.axis_index("c")
        s = lax.axis_index("s")
        wid = s * 2 + c
        pltpu.sync_copy(h_hbm.at[pl.ds(s * rows_t, rows_t)],
                        h_sh.at[pl.ds(s * rows_t, rows_t)])
        pltpu.sync_copy(dst_hbm.at[pl.ds(wid * per_w, per_w)], idxd_v)
        pltpu.sync_copy(src_hbm.at[pl.ds(wid * per_w, per_w)], idxs_v)
        plsc.subcore_barrier()
        slots = ((0, buf0, gsem0, ssem0), (1, buf1, gsem1, ssem1))
        for idx_v, out_hbm in ((idxd_v, oi_hbm), (idxs_v, oj_hbm)):
            def outer(o, _, idx_v=idx_v, out_hbm=out_hbm):
                for slot, buf, gsem, ssem in slots:
                    j = o * 2 + slot

                    @pl.when(o > 0)
                    def _(buf=buf, ssem=ssem):
                        pltpu.make_async_copy(
                            buf, out_hbm.at[pl.ds(0, GCH)], ssem).wait()

                    pltpu.async_copy(
                        h_sh.at[idx_v.at[pl.ds(j * GCH, GCH)]], buf, gsem)
                for slot, buf, gsem, ssem in slots:
                    j = o * 2 + slot
                    pltpu.make_async_copy(
                        h_sh.at[idx_v.at[pl.ds(j * GCH, GCH)]], buf,
                        gsem).wait()
                    pltpu.async_copy(
                        buf, out_hbm.at[pl.ds(wid * per_w + j * GCH, GCH)],
                        ssem)
                return 0

            lax.fori_loop(0, n_ch // 2, outer, 0)
            if n_ch % 2:
                j = n_ch - 1
                pltpu.make_async_copy(
                    buf0, out_hbm.at[pl.ds(0, GCH)], ssem0).wait()
                pltpu.async_copy(
                    h_sh.at[idx_v.at[pl.ds(j * GCH, GCH)]], buf0, gsem0)
                pltpu.make_async_copy(
                    h_sh.at[idx_v.at[pl.ds(j * GCH, GCH)]], buf0,
                    gsem0).wait()
                pltpu.async_copy(
                    buf0, out_hbm.at[pl.ds(wid * per_w + j * GCH, GCH)],
                    ssem0)
            for slot, buf, gsem, ssem in slots:
                pltpu.make_async_copy(
                    buf, out_hbm.at[pl.ds(0, GCH)], ssem).wait()

    return gk(h, dst, src)


def _sc_scatter(sm, sig, dst, zeros_nd):
    E, D = sm.shape
    NP = zeros_nd.shape[0]
    mesh = plsc.VectorSubcoreMesh(core_axis_name="c", subcore_axis_name="s")
    per_t = E // 16
    n_ch = per_t // SCH
    rows_t = NP // 16

    @functools.partial(
        pl.kernel,
        out_type=(jax.ShapeDtypeStruct((NP, D), jnp.float32),
                  jax.ShapeDtypeStruct((NP, D), jnp.float32)),
        mesh=mesh,
        scratch_types=[pltpu.VMEM((SCH,), jnp.int32),
                       pltpu.VMEM((SCH,), jnp.int32),
                       pltpu.VMEM((SCH, D), jnp.float32),
                       pltpu.VMEM((SCH, D), jnp.float32),
                       pltpu.VMEM_SHARED((NP, D), jnp.float32),
                       pltpu.SemaphoreType.DMA,
                       pltpu.SemaphoreType.DMA,
                       pltpu.SemaphoreType.DMA,
                       pltpu.SemaphoreType.DMA,
                       pltpu.SemaphoreType.DMA,
                       pltpu.SemaphoreType.DMA],
    )
    def sk(sm_hbm, sig_hbm, dst_hbm, z_hbm, num_hbm, den_hbm,
           ib0, ib1, vb0, vb1, acc_sh,
           isem0, isem1, vsem0, vsem1, asem0, asem1):
        c = lax.axis_index("c")
        s = lax.axis_index("s")
        pltpu.sync_copy(z_hbm.at[pl.ds(s * rows_t, rows_t)],
                        acc_sh.at[pl.ds(s * rows_t, rows_t)])
        plsc.subcore_barrier()
        slots = ((0, ib0, vb0, isem0, vsem0, asem0),
                 (1, ib1, vb1, isem1, vsem1, asem1))

        def outer(o, _):
            for slot, ib, vb, isem, vsem, asem in slots:
                j = o * 2 + slot
                base = s * per_t + j * SCH

                @pl.when(o > 0)
                def _(ib=ib, vb=vb, asem=asem):
                    pltpu.make_async_copy(vb, acc_sh.at[ib], asem).wait()

                pltpu.async_copy(dst_hbm.at[pl.ds(base, SCH)], ib, isem)

                @pl.when(c == 0)
                def _(vb=vb, vsem=vsem, base=base):
                    pltpu.async_copy(sm_hbm.at[pl.ds(base, SCH)], vb, vsem)

                @pl.when(c == 1)
                def _(vb=vb, vsem=vsem, base=base):
                    pltpu.async_copy(sig_hbm.at[pl.ds(base, SCH)], vb, vsem)

            for slot, ib, vb, isem, vsem, asem in slots:
                base = s * per_t + (o * 2 + slot) * SCH
                pltpu.make_async_copy(
                    dst_hbm.at[pl.ds(base, SCH)], ib, isem).wait()
                pltpu.make_async_copy(
                    sm_hbm.at[pl.ds(base, SCH)], vb, vsem).wait()
                pltpu.async_copy(vb, acc_sh.at[ib], asem, add=True)
            return 0

        lax.fori_loop(0, n_ch // 2, outer, 0)
        for slot, ib, vb, isem, vsem, asem in slots:
            pltpu.make_async_copy(vb, acc_sh.at[ib], asem).wait()
        plsc.subcore_barrier()

        @pl.when(c == 0)
        def _():
            pltpu.sync_copy(acc_sh.at[pl.ds(s * rows_t, rows_t)],
                            num_hbm.at[pl.ds(s * rows_t, rows_t)])

        @pl.when(c == 1)
        def _():
            pltpu.sync_copy(acc_sh.at[pl.ds(s * rows_t, rows_t)],
                            den_hbm.at[pl.ds(s * rows_t, rows_t)])

    return sk(sm, sig, dst, zeros_nd)


def _silu(v):
    return v / (1.0 + jnp.exp(-v))


def _tc_mlp(xi, xj, e, w1, b1, w2g, b2g, w2a, b2a):
    E, D = xi.shape
    n_blk = E // EB

    def body(xi_ref, xj_ref, e_ref, w1_ref, b1_ref, w2g_ref, b2g_ref,
             w2a_ref, b2a_ref, gate_ref, msg_ref, stats_ref, acc_ref):
        i = pl.program_id(0)

        @pl.when(i == 0)
        def _():
            acc_ref[...] = jnp.zeros_like(acc_ref)

        hid = (jnp.dot(xi_ref[...], w1_ref[0:D, :],
                       preferred_element_type=jnp.float32)
               + jnp.dot(xj_ref[...], w1_ref[D:2 * D, :],
                         preferred_element_type=jnp.float32)
               + jnp.dot(e_ref[...], w1_ref[2 * D:3 * D, :],
                         preferred_element_type=jnp.float32)
               + b1_ref[...])
        hid = _silu(hid)
        gate = jnp.dot(hid[:, :D], w2g_ref[...],
                       preferred_element_type=jnp.float32) + b2g_ref[...]
        msg = jnp.dot(hid[:, D:], w2a_ref[...],
                      preferred_element_type=jnp.float32) + b2a_ref[...]
        gate_ref[...] = gate
        msg_ref[...] = msg
        acc_ref[0:1, :] = acc_ref[0:1, :] + jnp.sum(gate, axis=0,
                                                    keepdims=True)
        acc_ref[1:2, :] = acc_ref[1:2, :] + jnp.sum(gate * gate, axis=0,
                                                    keepdims=True)
        stats_ref[...] = acc_ref[...]

    const = lambda i: (0, 0)
    return pl.pallas_call(
        body,
        grid=(n_blk,),
        in_specs=[
            pl.BlockSpec((EB, D), lambda i: (i, 0)),
            pl.BlockSpec((EB, D), lambda i: (i, 0)),
            pl.BlockSpec((EB, D), lambda i: (i, 0)),
            pl.BlockSpec((3 * D, 2 * D), const),
            pl.BlockSpec((1, 2 * D), const),
            pl.BlockSpec((D, D), const),
            pl.BlockSpec((1, D), const),
            pl.BlockSpec((D, D), const),
            pl.BlockSpec((1, D), const),
        ],
        out_specs=[
            pl.BlockSpec((EB, D), lambda i: (i, 0)),
            pl.BlockSpec((EB, D), lambda i: (i, 0)),
            pl.BlockSpec((8, D), const),
        ],
        out_shape=[
            jax.ShapeDtypeStruct((E, D), jnp.float32),
            jax.ShapeDtypeStruct((E, D), jnp.float32),
            jax.ShapeDtypeStruct((8, D), jnp.float32),
        ],
        scratch_shapes=[pltpu.VMEM((8, D), jnp.float32)],
    )(xi, xj, e, w1, b1, w2g, b2g, w2a, b2a)


def _tc_norm(gate_pre, msg, cart, stats, gamma, beta):
    E, D = gate_pre.shape
    n_blk = E // EB
    inv_e = 1.0 / E

    rows = EB // 128

    def body(g_ref, m_ref, c_ref, st_ref, ga_ref, be_ref,
             gn_ref, sm_ref, sg_ref):
        mu = st_ref[0:1, :] * inv_e
        var = st_ref[1:2, :] * inv_e - mu * mu
        rstd = lax.rsqrt(var + 1e-5)
        gate_n = (g_ref[...] - mu) * rstd * ga_ref[...] + be_ref[...]
        d = c_ref[0]
        env = 0.5 * (jnp.cos(jnp.pi * d / RADIUS) + 1.0)
        env = jnp.where(d < RADIUS, env, 0.0)
        ii = lax.broadcasted_iota(jnp.int32, (EB, rows), 0)
        rr = lax.broadcasted_iota(jnp.int32, (EB, rows), 1)
        sel = (ii // 128 == rr).astype(jnp.float32)
        x = jnp.dot(sel, env, preferred_element_type=jnp.float32)
        ic = lax.broadcasted_iota(jnp.int32, (EB, 128), 0) % 128
        cc = lax.broadcasted_iota(jnp.int32, (EB, 128), 1)
        env_col = jnp.sum(jnp.where(ic == cc, x, 0.0), axis=1,
                          keepdims=True)
        sig = env_col / (1.0 + jnp.exp(-gate_n))
        gn_ref[...] = gate_n
        sg_ref[...] = sig
        sm_ref[...] = sig * m_ref[...]

    const = lambda i: (0, 0)
    return pl.pallas_call(
        body,
        grid=(n_blk,),
        in_specs=[
            pl.BlockSpec((EB, D), lambda i: (i, 0)),
            pl.BlockSpec((EB, D), lambda i: (i, 0)),
            pl.BlockSpec((1, EB // 128, 128), lambda i: (i, 0, 0)),
            pl.BlockSpec((8, D), const),
            pl.BlockSpec((1, D), const),
            pl.BlockSpec((1, D), const),
        ],
        out_specs=[
            pl.BlockSpec((EB, D), lambda i: (i, 0)),
            pl.BlockSpec((EB, D), lambda i: (i, 0)),
            pl.BlockSpec((EB, D), lambda i: (i, 0)),
        ],
        out_shape=[
            jax.ShapeDtypeStruct((E, D), jnp.float32),
            jax.ShapeDtypeStruct((E, D), jnp.float32),
            jax.ShapeDtypeStruct((E, D), jnp.float32),
        ],
    )(gate_pre, msg, cart, stats, gamma, beta)


def _tc_node(num, den, h, bids_row, bids_col, alpha, gamma, beta):
    N, D = h.shape

    def body(num_ref, den_ref, h_ref, br_ref, bc_ref, al_ref, ga_ref, be_ref,
             out_ref):
        xnew = _silu(num_ref[...] / (den_ref[...] + 1e-6)) + h_ref[...]
        oh = (br_ref[...] == lax.broadcasted_iota(jnp.int32, (G, N), 0)
              ).astype(jnp.float32)
        oht = (bc_ref[...] == lax.broadcasted_iota(jnp.int32, (N, G), 1)
               ).astype(jnp.float32)
        cnt = jnp.maximum(jnp.sum(oh, axis=1, keepdims=True), 1.0)
        gmean = jnp.dot(oh, xnew, preferred_element_type=jnp.float32) / cnt
        xc = xnew - al_ref[...] * jnp.dot(
            oht, gmean, preferred_element_type=jnp.float32)
        gvar = jnp.dot(oh, xc * xc, preferred_element_type=jnp.float32) / cnt
        gv = jnp.dot(oht, gvar, preferred_element_type=jnp.float32)
        hn = xc * lax.rsqrt(gv + 1e-5) * ga_ref[...] + be_ref[...]
        out_ref[...] = h_ref[...] + hn

    const = lambda i: (0, 0)
    return pl.pallas_call(
        body,
        grid=(1,),
        in_specs=[
            pl.BlockSpec((N, D), const),
            pl.BlockSpec((N, D), const),
            pl.BlockSpec((N, D), const),
            pl.BlockSpec((1, N), const),
            pl.BlockSpec((N, 1), const),
            pl.BlockSpec((1, D), const),
            pl.BlockSpec((1, D), const),
            pl.BlockSpec((1, D), const),
        ],
        out_specs=pl.BlockSpec((N, D), const),
        out_shape=jax.ShapeDtypeStruct((N, D), jnp.float32),
    )(num, den, h, bids_row, bids_col, alpha, gamma, beta)


def kernel(x, edge_attr, cart_dist, gate_W1, gate_b1, gate_W2, gate_b2,
           aggr_W1, aggr_b1, aggr_W2, aggr_b2, bn_gamma, bn_beta,
           gn_alpha, gn_gamma, gn_beta, edge_index, batch_ids):
    N, D = x.shape
    E = edge_attr.shape[0]
    L = gate_W1.shape[0]
    src = edge_index[0]
    dst = edge_index[1]
    cart = cart_dist.reshape(E // EB, EB // 128, 128)
    NP = ((N + 127) // 128) * 128
    bids_p = jnp.pad(batch_ids, (0, NP - N), constant_values=G)
    bids_row = bids_p.reshape(1, NP)
    bids_col = bids_p.reshape(NP, 1)
    zeros_nd = jnp.zeros((NP, D), jnp.float32)

    h = jnp.pad(x, ((0, NP - N), (0, 0)))
    e = edge_attr
    for l in range(L):
        w1 = jnp.concatenate([gate_W1[l], aggr_W1[l]], axis=1)
        b1 = jnp.concatenate([gate_b1[l], aggr_b1[l]])[None, :]
        xi, xj = _sc_gather(h, dst, src)
        gate_pre, msg, stats = _tc_mlp(
            xi, xj, e, w1, b1,
            gate_W2[l], gate_b2[l][None, :], aggr_W2[l], aggr_b2[l][None, :])
        gate_n, sm, sig = _tc_norm(
            gate_pre, msg, cart, stats,
            bn_gamma[l][None, :], bn_beta[l][None, :])
        num, den = _sc_scatter(sm, sig, dst, zeros_nd)
        h = _tc_node(num, den, h, bids_row, bids_col,
                     gn_alpha[l][None, :], gn_gamma[l][None, :],
                     gn_beta[l][None, :])
        e = gate_n
    return h[:N]


# speedup vs baseline: 3.7957x; 1.0701x over previous
import functools

import jax
import jax.numpy as jnp
from jax import lax
from jax.experimental import pallas as pl
from jax.experimental.pallas import tpu as pltpu
from jax.experimental.pallas import tpu_sc as plsc

RADIUS = 5.0
G = 64
EB = 2560
GCH = 80
SCH = 80


def _sc_gather(h, dst, src):
    NP, D = h.shape
    E = dst.shape[0]
    mesh = plsc.VectorSubcoreMesh(core_axis_name="c", subcore_axis_name="s")
    per_w = E // 32
    n_ch = per_w // GCH
    rows_t = NP // 16

    @functools.partial(
        pl.kernel,
        out_type=(jax.ShapeDtypeStruct((E, D), jnp.float32),
                  jax.ShapeDtypeStruct((E, D), jnp.float32)),
        mesh=mesh,
        scratch_types=[pltpu.VMEM((per_w,), jnp.int32),
                       pltpu.VMEM((per_w,), jnp.int32),
                       pltpu.VMEM((GCH, D), jnp.float32),
                       pltpu.VMEM((GCH, D), jnp.float32),
                       pltpu.VMEM_SHARED((NP, D), jnp.float32),
                       pltpu.SemaphoreType.DMA,
                       pltpu.SemaphoreType.DMA,
                       pltpu.SemaphoreType.DMA,
                       pltpu.SemaphoreType.DMA],
    )
    def gk(h_hbm, dst_hbm, src_hbm, oi_hbm, oj_hbm,
           idxd_v, idxs_v, buf0, buf1, h_sh, gsem0, gsem1, ssem0, ssem1):
        c = lax.axis_index("c")
        s = lax.axis_index("s")
        wid = s * 2 + c
        pltpu.sync_copy(h_hbm.at[pl.ds(s * rows_t, rows_t)],
                        h_sh.at[pl.ds(s * rows_t, rows_t)])
        pltpu.sync_copy(dst_hbm.at[pl.ds(wid * per_w, per_w)], idxd_v)
        pltpu.sync_copy(src_hbm.at[pl.ds(wid * per_w, per_w)], idxs_v)
        plsc.subcore_barrier()
        slots = ((0, buf0, gsem0, ssem0), (1, buf1, gsem1, ssem1))
        for idx_v, out_hbm in ((idxd_v, oi_hbm), (idxs_v, oj_hbm)):
            def outer(o, _, idx_v=idx_v, out_hbm=out_hbm):
                for slot, buf, gsem, ssem in slots:
                    j = o * 2 + slot

                    @pl.when(o > 0)
                    def _(buf=buf, ssem=ssem):
                        pltpu.make_async_copy(
                            buf, out_hbm.at[pl.ds(0, GCH)], ssem).wait()

                    pltpu.async_copy(
                        h_sh.at[idx_v.at[pl.ds(j * GCH, GCH)]], buf, gsem)
                for slot, buf, gsem, ssem in slots:
                    j = o * 2 + slot
                    pltpu.make_async_copy(
                        h_sh.at[idx_v.at[pl.ds(j * GCH, GCH)]], buf,
                        gsem).wait()
                    pltpu.async_copy(
                        buf, out_hbm.at[pl.ds(wid * per_w + j * GCH, GCH)],
                        ssem)
                return 0

            lax.fori_loop(0, n_ch // 2, outer, 0)
            if n_ch % 2:
                j = n_ch - 1
                pltpu.make_async_copy(
                    buf0, out_hbm.at[pl.ds(0, GCH)], ssem0).wait()
                pltpu.async_copy(
                    h_sh.at[idx_v.at[pl.ds(j * GCH, GCH)]], buf0, gsem0)
                pltpu.make_async_copy(
                    h_sh.at[idx_v.at[pl.ds(j * GCH, GCH)]], buf0,
                    gsem0).wait()
                pltpu.async_copy(
                    buf0, out_hbm.at[pl.ds(wid * per_w + j * GCH, GCH)],
                    ssem0)
            for slot, buf, gsem, ssem in slots:
                pltpu.make_async_copy(
                    buf, out_hbm.at[pl.ds(0, GCH)], ssem).wait()

    return gk(h, dst, src)


def _sc_scatter(sm, sig, dst, zeros_nd):
    E, D = sm.shape
    NP = zeros_nd.shape[0]
    mesh = plsc.VectorSubcoreMesh(core_axis_name="c", subcore_axis_name="s")
    per_t = E // 16
    n_ch = per_t // SCH
    rows_t = NP // 16

    @functools.partial(
        pl.kernel,
        out_type=(jax.ShapeDtypeStruct((NP, D), jnp.float32),
                  jax.ShapeDtypeStruct((NP, D), jnp.float32)),
        mesh=mesh,
        scratch_types=[pltpu.VMEM((SCH,), jnp.int32),
                       pltpu.VMEM((SCH,), jnp.int32),
                       pltpu.VMEM((SCH, D), jnp.float32),
                       pltpu.VMEM((SCH, D), jnp.float32),
                       pltpu.VMEM_SHARED((NP, D), jnp.float32),
                       pltpu.SemaphoreType.DMA,
                       pltpu.SemaphoreType.DMA,
                       pltpu.SemaphoreType.DMA,
                       pltpu.SemaphoreType.DMA,
                       pltpu.SemaphoreType.DMA,
                       pltpu.SemaphoreType.DMA],
    )
    def sk(sm_hbm, sig_hbm, dst_hbm, z_hbm, num_hbm, den_hbm,
           ib0, ib1, vb0, vb1, acc_sh,
           isem0, isem1, vsem0, vsem1, asem0, asem1):
        c = lax.axis_index("c")
        s = lax.axis_index("s")
        pltpu.sync_copy(z_hbm.at[pl.ds(s * rows_t, rows_t)],
                        acc_sh.at[pl.ds(s * rows_t, rows_t)])
        plsc.subcore_barrier()
        slots = ((0, ib0, vb0, isem0, vsem0, asem0),
                 (1, ib1, vb1, isem1, vsem1, asem1))

        def outer(o, _):
            for slot, ib, vb, isem, vsem, asem in slots:
                j = o * 2 + slot
                base = s * per_t + j * SCH

                @pl.when(o > 0)
                def _(ib=ib, vb=vb, asem=asem):
                    pltpu.make_async_copy(vb, acc_sh.at[ib], asem).wait()

                pltpu.async_copy(dst_hbm.at[pl.ds(base, SCH)], ib, isem)

                @pl.when(c == 0)
                def _(vb=vb, vsem=vsem, base=base):
                    pltpu.async_copy(sm_hbm.at[pl.ds(base, SCH)], vb, vsem)

                @pl.when(c == 1)
                def _(vb=vb, vsem=vsem, base=base):
                    pltpu.async_copy(sig_hbm.at[pl.ds(base, SCH)], vb, vsem)

            for slot, ib, vb, isem, vsem, asem in slots:
                base = s * per_t + (o * 2 + slot) * SCH
                pltpu.make_async_copy(
                    dst_hbm.at[pl.ds(base, SCH)], ib, isem).wait()
                pltpu.make_async_copy(
                    sm_hbm.at[pl.ds(base, SCH)], vb, vsem).wait()
                pltpu.async_copy(vb, acc_sh.at[ib], asem, add=True)
            return 0

        lax.fori_loop(0, n_ch // 2, outer, 0)
        for slot, ib, vb, isem, vsem, asem in slots:
            pltpu.make_async_copy(vb, acc_sh.at[ib], asem).wait()
        plsc.subcore_barrier()

        @pl.when(c == 0)
        def _():
            pltpu.sync_copy(acc_sh.at[pl.ds(s * rows_t, rows_t)],
                            num_hbm.at[pl.ds(s * rows_t, rows_t)])

        @pl.when(c == 1)
        def _():
            pltpu.sync_copy(acc_sh.at[pl.ds(s * rows_t, rows_t)],
                            den_hbm.at[pl.ds(s * rows_t, rows_t)])

    return sk(sm, sig, dst, zeros_nd)


def _silu(v):
    return v / (1.0 + jnp.exp(-v))


def _tc_mlp(xi, xj, e, w1, b1, w2g, b2g, w2a, b2a):
    E, D = xi.shape
    n_blk = E // EB

    def body(xi_ref, xj_ref, e_ref, w1_ref, b1_ref, w2g_ref, b2g_ref,
             w2a_ref, b2a_ref, gate_ref, msg_ref, stats_ref, acc_ref):
        i = pl.program_id(0)

        @pl.when(i == 0)
        def _():
            acc_ref[...] = jnp.zeros_like(acc_ref)

        w1b = w1_ref[...].astype(jnp.bfloat16)
        hid = (jnp.dot(xi_ref[...].astype(jnp.bfloat16), w1b[0:D, :],
                       preferred_element_type=jnp.float32)
               + jnp.dot(xj_ref[...].astype(jnp.bfloat16), w1b[D:2 * D, :],
                         preferred_element_type=jnp.float32)
               + jnp.dot(e_ref[...], w1b[2 * D:3 * D, :],
                         preferred_element_type=jnp.float32)
               + b1_ref[...])
        hid = _silu(hid).astype(jnp.bfloat16)
        gate = jnp.dot(hid[:, :D], w2g_ref[...].astype(jnp.bfloat16),
                       preferred_element_type=jnp.float32) + b2g_ref[...]
        msg = jnp.dot(hid[:, D:], w2a_ref[...].astype(jnp.bfloat16),
                      preferred_element_type=jnp.float32) + b2a_ref[...]
        gate_ref[...] = gate.astype(jnp.bfloat16)
        msg_ref[...] = msg.astype(jnp.bfloat16)
        acc_ref[0:1, :] = acc_ref[0:1, :] + jnp.sum(gate, axis=0,
                                                    keepdims=True)
        acc_ref[1:2, :] = acc_ref[1:2, :] + jnp.sum(gate * gate, axis=0,
                                                    keepdims=True)
        stats_ref[...] = acc_ref[...]

    const = lambda i: (0, 0)
    return pl.pallas_call(
        body,
        grid=(n_blk,),
        in_specs=[
            pl.BlockSpec((EB, D), lambda i: (i, 0)),
            pl.BlockSpec((EB, D), lambda i: (i, 0)),
            pl.BlockSpec((EB, D), lambda i: (i, 0)),
            pl.BlockSpec((3 * D, 2 * D), const),
            pl.BlockSpec((1, 2 * D), const),
            pl.BlockSpec((D, D), const),
            pl.BlockSpec((1, D), const),
            pl.BlockSpec((D, D), const),
            pl.BlockSpec((1, D), const),
        ],
        out_specs=[
            pl.BlockSpec((EB, D), lambda i: (i, 0)),
            pl.BlockSpec((EB, D), lambda i: (i, 0)),
            pl.BlockSpec((8, D), const),
        ],
        out_shape=[
            jax.ShapeDtypeStruct((E, D), jnp.bfloat16),
            jax.ShapeDtypeStruct((E, D), jnp.bfloat16),
            jax.ShapeDtypeStruct((8, D), jnp.float32),
        ],
        scratch_shapes=[pltpu.VMEM((8, D), jnp.float32)],
    )(xi, xj, e, w1, b1, w2g, b2g, w2a, b2a)


def _tc_norm(gate_pre, msg, cart, stats, gamma, beta):
    E, D = gate_pre.shape
    n_blk = E // EB
    inv_e = 1.0 / E

    rows = EB // 128

    def body(g_ref, m_ref, c_ref, st_ref, ga_ref, be_ref,
             gn_ref, sm_ref, sg_ref):
        mu = st_ref[0:1, :] * inv_e
        var = st_ref[1:2, :] * inv_e - mu * mu
        rstd = lax.rsqrt(var + 1e-5)
        gate_n = ((g_ref[...].astype(jnp.float32) - mu) * rstd
                  * ga_ref[...] + be_ref[...])
        d = c_ref[0]
        env = 0.5 * (jnp.cos(jnp.pi * d / RADIUS) + 1.0)
        env = jnp.where(d < RADIUS, env, 0.0)
        ii = lax.broadcasted_iota(jnp.int32, (EB, rows), 0)
        rr = lax.broadcasted_iota(jnp.int32, (EB, rows), 1)
        sel = (ii // 128 == rr).astype(jnp.float32)
        x = jnp.dot(sel, env, preferred_element_type=jnp.float32)
        ic = lax.broadcasted_iota(jnp.int32, (EB, 128), 0) % 128
        cc = lax.broadcasted_iota(jnp.int32, (EB, 128), 1)
        env_col = jnp.sum(jnp.where(ic == cc, x, 0.0), axis=1,
                          keepdims=True)
        sig = env_col / (1.0 + jnp.exp(-gate_n))
        gn_ref[...] = gate_n.astype(jnp.bfloat16)
        sg_ref[...] = sig
        sm_ref[...] = sig * m_ref[...].astype(jnp.float32)

    const = lambda i: (0, 0)
    return pl.pallas_call(
        body,
        grid=(n_blk,),
        in_specs=[
            pl.BlockSpec((EB, D), lambda i: (i, 0)),
            pl.BlockSpec((EB, D), lambda i: (i, 0)),
            pl.BlockSpec((1, EB // 128, 128), lambda i: (i, 0, 0)),
            pl.BlockSpec((8, D), const),
            pl.BlockSpec((1, D), const),
            pl.BlockSpec((1, D), const),
        ],
        out_specs=[
            pl.BlockSpec((EB, D), lambda i: (i, 0)),
            pl.BlockSpec((EB, D), lambda i: (i, 0)),
            pl.BlockSpec((EB, D), lambda i: (i, 0)),
        ],
        out_shape=[
            jax.ShapeDtypeStruct((E, D), jnp.bfloat16),
            jax.ShapeDtypeStruct((E, D), jnp.float32),
            jax.ShapeDtypeStruct((E, D), jnp.float32),
        ],
    )(gate_pre, msg, cart, stats, gamma, beta)


def _tc_node(num, den, h, bids_row, bids_col, alpha, gamma, beta):
    N, D = h.shape

    def body(num_ref, den_ref, h_ref, br_ref, bc_ref, al_ref, ga_ref, be_ref,
             out_ref):
        xnew = _silu(num_ref[...] / (den_ref[...] + 1e-6)) + h_ref[...]
        oh = (br_ref[...] == lax.broadcasted_iota(jnp.int32, (G, N), 0)
              ).astype(jnp.float32)
        oht = (bc_ref[...] == lax.broadcasted_iota(jnp.int32, (N, G), 1)
               ).astype(jnp.float32)
        cnt = jnp.maximum(jnp.sum(oh, axis=1, keepdims=True), 1.0)
        gmean = jnp.dot(oh, xnew, preferred_element_type=jnp.float32) / cnt
        xc = xnew - al_ref[...] * jnp.dot(
            oht, gmean, preferred_element_type=jnp.float32)
        gvar = jnp.dot(oh, xc * xc, preferred_element_type=jnp.float32) / cnt
        gv = jnp.dot(oht, gvar, preferred_element_type=jnp.float32)
        hn = xc * lax.rsqrt(gv + 1e-5) * ga_ref[...] + be_ref[...]
        out_ref[...] = h_ref[...] + hn

    const = lambda i: (0, 0)
    return pl.pallas_call(
        body,
        grid=(1,),
        in_specs=[
            pl.BlockSpec((N, D), const),
            pl.BlockSpec((N, D), const),
            pl.BlockSpec((N, D), const),
            pl.BlockSpec((1, N), const),
            pl.BlockSpec((N, 1), const),
            pl.BlockSpec((1, D), const),
            pl.BlockSpec((1, D), const),
            pl.BlockSpec((1, D), const),
        ],
        out_specs=pl.BlockSpec((N, D), const),
        out_shape=jax.ShapeDtypeStruct((N, D), jnp.float32),
    )(num, den, h, bids_row, bids_col, alpha, gamma, beta)


def kernel(x, edge_attr, cart_dist, gate_W1, gate_b1, gate_W2, gate_b2,
           aggr_W1, aggr_b1, aggr_W2, aggr_b2, bn_gamma, bn_beta,
           gn_alpha, gn_gamma, gn_beta, edge_index, batch_ids):
    N, D = x.shape
    E = edge_attr.shape[0]
    L = gate_W1.shape[0]
    src = edge_index[0]
    dst = edge_index[1]
    cart = cart_dist.reshape(E // EB, EB // 128, 128)
    NP = ((N + 127) // 128) * 128
    bids_p = jnp.pad(batch_ids, (0, NP - N), constant_values=G)
    bids_row = bids_p.reshape(1, NP)
    bids_col = bids_p.reshape(NP, 1)
    zeros_nd = jnp.zeros((NP, D), jnp.float32)

    h = jnp.pad(x, ((0, NP - N), (0, 0)))
    e = edge_attr.astype(jnp.bfloat16)
    for l in range(L):
        w1 = jnp.concatenate([gate_W1[l], aggr_W1[l]], axis=1)
        b1 = jnp.concatenate([gate_b1[l], aggr_b1[l]])[None, :]
        xi, xj = _sc_gather(h, dst, src)
        gate_pre, msg, stats = _tc_mlp(
            xi, xj, e, w1, b1,
            gate_W2[l], gate_b2[l][None, :], aggr_W2[l], aggr_b2[l][None, :])
        gate_n, sm, sig = _tc_norm(
            gate_pre, msg, cart, stats,
            bn_gamma[l][None, :], bn_beta[l][None, :])
        num, den = _sc_scatter(sm, sig, dst, zeros_nd)
        h = _tc_node(num, den, h, bids_row, bids_col,
                     gn_alpha[l][None, :], gn_gamma[l][None, :],
                     gn_beta[l][None, :])
        e = gate_n
    return h[:N]


# speedup vs baseline: 3.9297x; 1.0353x over previous
import functools

import jax
import jax.numpy as jnp
from jax import lax
from jax.experimental import pallas as pl
from jax.experimental.pallas import tpu as pltpu
from jax.experimental.pallas import tpu_sc as plsc

RADIUS = 5.0
G = 64
EB = 2560
GCH = 80
SCH = 160


def _sc_gather(h, dst, src):
    NP, D = h.shape
    E = dst.shape[0]
    mesh = plsc.VectorSubcoreMesh(core_axis_name="c", subcore_axis_name="s")
    per_w = E // 32
    n_ch = per_w // GCH
    rows_t = NP // 16

    @functools.partial(
        pl.kernel,
        out_type=(jax.ShapeDtypeStruct((E, D), jnp.float32),
                  jax.ShapeDtypeStruct((E, D), jnp.float32)),
        mesh=mesh,
        scratch_types=[pltpu.VMEM((per_w,), jnp.int32),
                       pltpu.VMEM((per_w,), jnp.int32),
                       pltpu.VMEM((GCH, D), jnp.float32),
                       pltpu.VMEM((GCH, D), jnp.float32),
                       pltpu.VMEM_SHARED((NP, D), jnp.float32),
                       pltpu.SemaphoreType.DMA,
                       pltpu.SemaphoreType.DMA,
                       pltpu.SemaphoreType.DMA,
                       pltpu.SemaphoreType.DMA],
    )
    def gk(h_hbm, dst_hbm, src_hbm, oi_hbm, oj_hbm,
           idxd_v, idxs_v, buf0, buf1, h_sh, gsem0, gsem1, ssem0, ssem1):
        c = lax.axis_index("c")
        s = lax.axis_index("s")
        wid = s * 2 + c
        pltpu.sync_copy(h_hbm.at[pl.ds(s * rows_t, rows_t)],
                        h_sh.at[pl.ds(s * rows_t, rows_t)])
        pltpu.sync_copy(dst_hbm.at[pl.ds(wid * per_w, per_w)], idxd_v)
        pltpu.sync_copy(src_hbm.at[pl.ds(wid * per_w, per_w)], idxs_v)
        plsc.subcore_barrier()
        slots = ((0, buf0, gsem0, ssem0), (1, buf1, gsem1, ssem1))
        for idx_v, out_hbm in ((idxd_v, oi_hbm), (idxs_v, oj_hbm)):
            def outer(o, _, idx_v=idx_v, out_hbm=out_hbm):
                for slot, buf, gsem, ssem in slots:
                    j = o * 2 + slot

                    @pl.when(o > 0)
                    def _(buf=buf, ssem=ssem):
                        pltpu.make_async_copy(
                            buf, out_hbm.at[pl.ds(0, GCH)], ssem).wait()

                    pltpu.async_copy(
                        h_sh.at[idx_v.at[pl.ds(j * GCH, GCH)]], buf, gsem)
                for slot, buf, gsem, ssem in slots:
                    j = o * 2 + slot
                    pltpu.make_async_copy(
                        h_sh.at[idx_v.at[pl.ds(j * GCH, GCH)]], buf,
                        gsem).wait()
                    pltpu.async_copy(
                        buf, out_hbm.at[pl.ds(wid * per_w + j * GCH, GCH)],
                        ssem)
                return 0

            lax.fori_loop(0, n_ch // 2, outer, 0)
            if n_ch % 2:
                j = n_ch - 1
                pltpu.make_async_copy(
                    buf0, out_hbm.at[pl.ds(0, GCH)], ssem0).wait()
                pltpu.async_copy(
                    h_sh.at[idx_v.at[pl.ds(j * GCH, GCH)]], buf0, gsem0)
                pltpu.make_async_copy(
                    h_sh.at[idx_v.at[pl.ds(j * GCH, GCH)]], buf0,
                    gsem0).wait()
                pltpu.async_copy(
                    buf0, out_hbm.at[pl.ds(wid * per_w + j * GCH, GCH)],
                    ssem0)
            for slot, buf, gsem, ssem in slots:
                pltpu.make_async_copy(
                    buf, out_hbm.at[pl.ds(0, GCH)], ssem).wait()

    return gk(h, dst, src)


def _sc_scatter(sm, sig, dst, zeros_nd):
    E, D = sm.shape
    NP = zeros_nd.shape[0]
    mesh = plsc.VectorSubcoreMesh(core_axis_name="c", subcore_axis_name="s")
    per_t = E // 16
    n_ch = per_t // SCH
    rows_t = NP // 16

    @functools.partial(
        pl.kernel,
        out_type=(jax.ShapeDtypeStruct((NP, D), jnp.float32),
                  jax.ShapeDtypeStruct((NP, D), jnp.float32)),
        mesh=mesh,
        scratch_types=[pltpu.VMEM((SCH,), jnp.int32),
                       pltpu.VMEM((SCH,), jnp.int32),
                       pltpu.VMEM((SCH, D), jnp.float32),
                       pltpu.VMEM((SCH, D), jnp.float32),
                       pltpu.VMEM_SHARED((NP, D), jnp.float32),
                       pltpu.SemaphoreType.DMA,
                       pltpu.SemaphoreType.DMA,
                       pltpu.SemaphoreType.DMA,
                       pltpu.SemaphoreType.DMA,
                       pltpu.SemaphoreType.DMA,
                       pltpu.SemaphoreType.DMA],
    )
    def sk(sm_hbm, sig_hbm, dst_hbm, z_hbm, num_hbm, den_hbm,
           ib0, ib1, vb0, vb1, acc_sh,
           isem0, isem1, vsem0, vsem1, asem0, asem1):
        c = lax.axis_index("c")
        s = lax.axis_index("s")
        pltpu.sync_copy(z_hbm.at[pl.ds(s * rows_t, rows_t)],
                        acc_sh.at[pl.ds(s * rows_t, rows_t)])
        plsc.subcore_barrier()
        slots = ((0, ib0, vb0, isem0, vsem0, asem0),
                 (1, ib1, vb1, isem1, vsem1, asem1))

        def outer(o, _):
            for slot, ib, vb, isem, vsem, asem in slots:
                j = o * 2 + slot
                base = s * per_t + j * SCH

                @pl.when(o > 0)
                def _(ib=ib, vb=vb, asem=asem):
                    pltpu.make_async_copy(vb, acc_sh.at[ib], asem).wait()

                pltpu.async_copy(dst_hbm.at[pl.ds(base, SCH)], ib, isem)

                @pl.when(c == 0)
                def _(vb=vb, vsem=vsem, base=base):
                    pltpu.async_copy(sm_hbm.at[pl.ds(base, SCH)], vb, vsem)

                @pl.when(c == 1)
                def _(vb=vb, vsem=vsem, base=base):
                    pltpu.async_copy(sig_hbm.at[pl.ds(base, SCH)], vb, vsem)

            for slot, ib, vb, isem, vsem, asem in slots:
                base = s * per_t + (o * 2 + slot) * SCH
                pltpu.make_async_copy(
                    dst_hbm.at[pl.ds(base, SCH)], ib, isem).wait()
                pltpu.make_async_copy(
                    sm_hbm.at[pl.ds(base, SCH)], vb, vsem).wait()
                pltpu.async_copy(vb, acc_sh.at[ib], asem, add=True)
            return 0

        lax.fori_loop(0, n_ch // 2, outer, 0)
        if n_ch % 2:
            j = n_ch - 1
            base = s * per_t + j * SCH
            pltpu.make_async_copy(vb0, acc_sh.at[ib0], asem0).wait()
            pltpu.sync_copy(dst_hbm.at[pl.ds(base, SCH)], ib0)

            @pl.when(c == 0)
            def _():
                pltpu.sync_copy(sm_hbm.at[pl.ds(base, SCH)], vb0)

            @pl.when(c == 1)
            def _():
                pltpu.sync_copy(sig_hbm.at[pl.ds(base, SCH)], vb0)

            pltpu.async_copy(vb0, acc_sh.at[ib0], asem0, add=True)
        for slot, ib, vb, isem, vsem, asem in slots:
            pltpu.make_async_copy(vb, acc_sh.at[ib], asem).wait()
        plsc.subcore_barrier()

        @pl.when(c == 0)
        def _():
            pltpu.sync_copy(acc_sh.at[pl.ds(s * rows_t, rows_t)],
                            num_hbm.at[pl.ds(s * rows_t, rows_t)])

        @pl.when(c == 1)
        def _():
            pltpu.sync_copy(acc_sh.at[pl.ds(s * rows_t, rows_t)],
                            den_hbm.at[pl.ds(s * rows_t, rows_t)])

    return sk(sm, sig, dst, zeros_nd)


def _silu(v):
    return v / (1.0 + jnp.exp(-v))


def _tc_mlp(xi, xj, e, w1, b1, w2g, b2g, w2a, b2a):
    E, D = e.shape
    n_blk = E // EB

    def body(xi_ref, xj_ref, e_ref, w1_ref, b1_ref, w2g_ref, b2g_ref,
             w2a_ref, b2a_ref, gate_ref, msg_ref, stats_ref, acc_ref):
        i = pl.program_id(0)

        @pl.when(i == 0)
        def _():
            acc_ref[...] = jnp.zeros_like(acc_ref)

        w1b = w1_ref[...].astype(jnp.bfloat16)
        hid = (jnp.dot(xi_ref[...].astype(jnp.bfloat16), w1b[0:D, :],
                       preferred_element_type=jnp.float32)
               + jnp.dot(xj_ref[...].astype(jnp.bfloat16), w1b[D:2 * D, :],
                         preferred_element_type=jnp.float32)
               + jnp.dot(e_ref[...], w1b[2 * D:3 * D, :],
                         preferred_element_type=jnp.float32)
               + b1_ref[...])
        hid = _silu(hid).astype(jnp.bfloat16)
        gate = jnp.dot(hid[:, :D], w2g_ref[...].astype(jnp.bfloat16),
                       preferred_element_type=jnp.float32) + b2g_ref[...]
        msg = jnp.dot(hid[:, D:], w2a_ref[...].astype(jnp.bfloat16),
                      preferred_element_type=jnp.float32) + b2a_ref[...]
        gate_ref[...] = gate.astype(jnp.bfloat16)
        msg_ref[...] = msg.astype(jnp.bfloat16)
        acc_ref[0:1, :] = acc_ref[0:1, :] + jnp.sum(gate, axis=0,
                                                    keepdims=True)
        acc_ref[1:2, :] = acc_ref[1:2, :] + jnp.sum(gate * gate, axis=0,
                                                    keepdims=True)
        stats_ref[...] = acc_ref[...]

    const = lambda i: (0, 0)
    return pl.pallas_call(
        body,
        grid=(n_blk,),
        in_specs=[
            pl.BlockSpec((EB, D), lambda i: (i, 0)),
            pl.BlockSpec((EB, D), lambda i: (i, 0)),
            pl.BlockSpec((EB, D), lambda i: (i, 0)),
            pl.BlockSpec((3 * D, 2 * D), const),
            pl.BlockSpec((1, 2 * D), const),
            pl.BlockSpec((D, D), const),
            pl.BlockSpec((1, D), const),
            pl.BlockSpec((D, D), const),
            pl.BlockSpec((1, D), const),
        ],
        out_specs=[
            pl.BlockSpec((EB, D), lambda i: (i, 0)),
            pl.BlockSpec((EB, D), lambda i: (i, 0)),
            pl.BlockSpec((8, D), const),
        ],
        out_shape=[
            jax.ShapeDtypeStruct((E, D), jnp.bfloat16),
            jax.ShapeDtypeStruct((E, D), jnp.bfloat16),
            jax.ShapeDtypeStruct((8, D), jnp.float32),
        ],
        scratch_shapes=[pltpu.VMEM((8, D), jnp.float32)],
    )(xi, xj, e, w1, b1, w2g, b2g, w2a, b2a)


def _tc_norm(gate_pre, msg, cart, stats, gamma, beta, want_gn):
    E, D = gate_pre.shape
    n_blk = E // EB
    inv_e = 1.0 / E

    rows = EB // 128

    def body(g_ref, m_ref, c_ref, st_ref, ga_ref, be_ref,
             *out_refs):
        if want_gn:
            gn_ref, sm_ref, sg_ref = out_refs
        else:
            sm_ref, sg_ref = out_refs
        mu = st_ref[0:1, :] * inv_e
        var = st_ref[1:2, :] * inv_e - mu * mu
        rstd = lax.rsqrt(var + 1e-5)
        gate_n = ((g_ref[...].astype(jnp.float32) - mu) * rstd
                  * ga_ref[...] + be_ref[...])
        d = c_ref[0]
        env = 0.5 * (jnp.cos(jnp.pi * d / RADIUS) + 1.0)
        env = jnp.where(d < RADIUS, env, 0.0)
        ii = lax.broadcasted_iota(jnp.int32, (EB, rows), 0)
        rr = lax.broadcasted_iota(jnp.int32, (EB, rows), 1)
        sel = (ii // 128 == rr).astype(jnp.float32)
        x = jnp.dot(sel, env, preferred_element_type=jnp.float32)
        ic = lax.broadcasted_iota(jnp.int32, (EB, 128), 0) % 128
        cc = lax.broadcasted_iota(jnp.int32, (EB, 128), 1)
        env_col = jnp.sum(jnp.where(ic == cc, x, 0.0), axis=1,
                          keepdims=True)
        sig = env_col / (1.0 + jnp.exp(-gate_n))
        if want_gn:
            gn_ref[...] = gate_n.astype(jnp.bfloat16)
        sg_ref[...] = sig
        sm_ref[...] = sig * m_ref[...].astype(jnp.float32)

    const = lambda i: (0, 0)
    return pl.pallas_call(
        body,
        grid=(n_blk,),
        in_specs=[
            pl.BlockSpec((EB, D), lambda i: (i, 0)),
            pl.BlockSpec((EB, D), lambda i: (i, 0)),
            pl.BlockSpec((1, EB // 128, 128), lambda i: (i, 0, 0)),
            pl.BlockSpec((8, D), const),
            pl.BlockSpec((1, D), const),
            pl.BlockSpec((1, D), const),
        ],
        out_specs=[pl.BlockSpec((EB, D), lambda i: (i, 0))] * (
            3 if want_gn else 2),
        out_shape=([jax.ShapeDtypeStruct((E, D), jnp.bfloat16)]
                   if want_gn else [])
        + [jax.ShapeDtypeStruct((E, D), jnp.float32),
           jax.ShapeDtypeStruct((E, D), jnp.float32)],
    )(gate_pre, msg, cart, stats, gamma, beta)


def _tc_node(num, den, h, bids_row, bids_col, alpha, gamma, beta):
    N, D = h.shape

    def body(num_ref, den_ref, h_ref, br_ref, bc_ref, al_ref, ga_ref, be_ref,
             out_ref):
        xnew = _silu(num_ref[...] / (den_ref[...] + 1e-6)) + h_ref[...]
        oh = (br_ref[...] == lax.broadcasted_iota(jnp.int32, (G, N), 0)
              ).astype(jnp.float32)
        oht = (bc_ref[...] == lax.broadcasted_iota(jnp.int32, (N, G), 1)
               ).astype(jnp.float32)
        cnt = jnp.maximum(jnp.sum(oh, axis=1, keepdims=True), 1.0)
        gmean = jnp.dot(oh, xnew, preferred_element_type=jnp.float32) / cnt
        xc = xnew - al_ref[...] * jnp.dot(
            oht, gmean, preferred_element_type=jnp.float32)
        gvar = jnp.dot(oh, xc * xc, preferred_element_type=jnp.float32) / cnt
        gv = jnp.dot(oht, gvar, preferred_element_type=jnp.float32)
        hn = xc * lax.rsqrt(gv + 1e-5) * ga_ref[...] + be_ref[...]
        out_ref[...] = h_ref[...] + hn

    const = lambda i: (0, 0)
    return pl.pallas_call(
        body,
        grid=(1,),
        in_specs=[
            pl.BlockSpec((N, D), const),
            pl.BlockSpec((N, D), const),
            pl.BlockSpec((N, D), const),
            pl.BlockSpec((1, N), const),
            pl.BlockSpec((N, 1), const),
            pl.BlockSpec((1, D), const),
            pl.BlockSpec((1, D), const),
            pl.BlockSpec((1, D), const),
        ],
        out_specs=pl.BlockSpec((N, D), const),
        out_shape=jax.ShapeDtypeStruct((N, D), jnp.float32),
    )(num, den, h, bids_row, bids_col, alpha, gamma, beta)


def kernel(x, edge_attr, cart_dist, gate_W1, gate_b1, gate_W2, gate_b2,
           aggr_W1, aggr_b1, aggr_W2, aggr_b2, bn_gamma, bn_beta,
           gn_alpha, gn_gamma, gn_beta, edge_index, batch_ids):
    N, D = x.shape
    E = edge_attr.shape[0]
    L = gate_W1.shape[0]
    src = edge_index[0]
    dst = edge_index[1]
    cart = cart_dist.reshape(E // EB, EB // 128, 128)
    NP = ((N + 127) // 128) * 128
    bids_p = jnp.pad(batch_ids, (0, NP - N), constant_values=G)
    bids_row = bids_p.reshape(1, NP)
    bids_col = bids_p.reshape(NP, 1)
    zeros_nd = jnp.zeros((NP, D), jnp.float32)

    h = jnp.pad(x, ((0, NP - N), (0, 0)))
    e = edge_attr.astype(jnp.bfloat16)
    for l in range(L):
        w1 = jnp.concatenate([gate_W1[l], aggr_W1[l]], axis=1)
        b1 = jnp.concatenate([gate_b1[l], aggr_b1[l]])[None, :]
        xi, xj = _sc_gather(h, dst, src)
        gate_pre, msg, stats = _tc_mlp(
            xi, xj, e, w1, b1,
            gate_W2[l], gate_b2[l][None, :], aggr_W2[l], aggr_b2[l][None, :])
        want_gn = l + 1 < L
        outs = _tc_norm(gate_pre, msg, cart, stats,
                        bn_gamma[l][None, :], bn_beta[l][None, :], want_gn)
        if want_gn:
            gate_n, sm, sig = outs
            e = gate_n
        else:
            sm, sig = outs
        num, den = _sc_scatter(sm, sig, dst, zeros_nd)
        h = _tc_node(num, den, h, bids_row, bids_col,
                     gn_alpha[l][None, :], gn_gamma[l][None, :],
                     gn_beta[l][None, :])
    return h[:N]
